# baseline (device time: 74746 ns/iter reference)
import functools

import jax
import jax.numpy as jnp
from jax import lax
from jax.experimental import pallas as pl
from jax.experimental.pallas import tpu as pltpu


def kernel(x, W):
    t, d = x.shape
    _, v_half = W.shape
    CH = 512
    NCH = v_half // CH
    QC = NCH // 4
    NS = 4
    f32 = jnp.float32
    PERM = [q * QC + j for j in range(QC) for q in range(4)]

    def body(x_ref, w_hbm, out_hbm, logits_ref, comm_ref, wbuf, ostage,
             stats_tx, stats_rx, wsems, osems, send_y, recv_y, sx1, rx1,
             sz1, rz1, sx2, rx2, sz2, rz2, stats_ssem, stats_rsem):
        my_x = lax.axis_index("x")
        my_y = lax.axis_index("y")
        my_z = lax.axis_index("z")
        ypeer = (my_x, 1 - my_y, my_z)
        xnbr = (1 - my_x, my_y, my_z)
        znbr = (my_x, my_y, 1 - my_z)
        own_base = my_y * v_half
        oth_base = (1 - my_y) * v_half
        q_mine = 2 * my_x + my_z
        q_x = 2 * (1 - my_x) + my_z
        q_z = 2 * my_x + (1 - my_z)
        q_d = 2 * (1 - my_x) + (1 - my_z)

        barrier = pltpu.get_barrier_semaphore()
        for nbr in (ypeer, xnbr, znbr):
            pl.semaphore_signal(
                barrier, inc=1, device_id=nbr,
                device_id_type=pl.DeviceIdType.MESH,
            )
        pl.semaphore_wait(barrier, 3)

        xb = x_ref[...].astype(jnp.bfloat16)

        def w_copy(i, k):
            return pltpu.make_async_copy(
                w_hbm.at[:, pl.ds(k * CH, CH)], wbuf.at[i % 2], wsems.at[i % 2]
            )

        def rdma(idx, dev, ssem, rsem):
            return pltpu.make_async_remote_copy(
                src_ref=comm_ref.at[idx],
                dst_ref=comm_ref.at[idx],
                send_sem=ssem,
                recv_sem=rsem,
                device_id=dev,
                device_id_type=pl.DeviceIdType.MESH,
            )

        y_rdmas = [
            pltpu.make_async_remote_copy(
                src_ref=logits_ref.at[q_mine * QC + j],
                dst_ref=comm_ref.at[q_mine * QC + j],
                send_sem=send_y.at[j],
                recv_sem=recv_y.at[j],
                device_id=ypeer,
                device_id_type=pl.DeviceIdType.MESH,
            )
            for j in range(QC)
        ]
        x1_out = [rdma(q_mine * QC + j, xnbr, sx1.at[j], rx1.at[j])
                  for j in range(QC)]
        z1_out = [rdma(q_mine * QC + j, znbr, sz1.at[j], rz1.at[j])
                  for j in range(QC)]
        x1_in = [rdma(q_x * QC + j, xnbr, sx1.at[j], rx1.at[j])
                 for j in range(QC)]
        z1_in = [rdma(q_z * QC + j, znbr, sz1.at[j], rz1.at[j])
                 for j in range(QC)]
        x2_out = [rdma(q_z * QC + j, xnbr, sx2.at[j], rx2.at[j])
                  for j in range(2)]
        z2_out = [rdma(q_x * QC + 2 + j, znbr, sz2.at[j], rz2.at[j])
                  for j in range(2)]
        x2_in = [rdma(q_d * QC + j, xnbr, sx2.at[j], rx2.at[j])
                 for j in range(2)]
        z2_in = [rdma(q_d * QC + 2 + j, znbr, sz2.at[j], rz2.at[j])
                 for j in range(2)]

        def relay(j):
            y_rdmas[j].wait_recv()
            x1_out[j].start()
            z1_out[j].start()

        w_copy(0, PERM[0]).start()
        m_parts = []
        for i, k in enumerate(PERM):
            if i + 1 < NCH:
                w_copy(i + 1, PERM[i + 1]).start()
            w_copy(i, k).wait()
            acc = jnp.dot(
                xb, wbuf[i % 2].astype(jnp.bfloat16),
                preferred_element_type=f32,
            )
            logits_ref[k] = acc.astype(jnp.bfloat16)
            m_parts.append(acc.max(axis=-1, keepdims=True))

            @pl.when(q_mine == k // QC)
            def _(j=k % QC):
                y_rdmas[j].start()

            if i == 8:
                relay(0)
            elif i == 12:
                relay(1)

        relay(2)
        relay(3)
        for j in range(2):
            z1_in[j].wait_recv()
            x2_out[j].start()
        for j in range(2):
            x1_in[2 + j].wait_recv()
            z2_out[j].start()

        m_own = functools.reduce(jnp.maximum, m_parts)
        s_own = None
        for k in range(NCH):
            es = jnp.sum(
                jnp.exp(logits_ref[k].astype(f32) - m_own),
                axis=-1, keepdims=True,
            )
            s_own = es if s_own is None else s_own + es
        stats_tx[:, 0:1] = m_own
        stats_tx[:, 1:2] = s_own
        srdma = pltpu.make_async_remote_copy(
            src_ref=stats_tx,
            dst_ref=stats_rx,
            send_sem=stats_ssem,
            recv_sem=stats_rsem,
            device_id=ypeer,
            device_id_type=pl.DeviceIdType.MESH,
        )
        srdma.start()

        srdma.wait_recv()
        m_oth = stats_rx[:, 0:1]
        s_oth = stats_rx[:, 1:2]
        m = jnp.maximum(m_own, m_oth)
        r = 1.0 / (s_own * jnp.exp(m_own - m) + s_oth * jnp.exp(m_oth - m))

        ocopies = [None] * NS
        ordinal = [0]

        def out_store(col, val):
            slot = ordinal[0] % NS
            if ocopies[slot] is not None:
                ocopies[slot].wait()
            ostage[slot] = val.astype(ostage.dtype)
            cp = pltpu.make_async_copy(
                ostage.at[slot], out_hbm.at[:, pl.ds(col, CH)], osems.at[slot]
            )
            cp.start()
            ocopies[slot] = cp
            ordinal[0] += 1

        for k in range(NCH):
            val = jnp.exp(logits_ref[k].astype(f32) - m) * r
            out_store(own_base + k * CH, val)
        for c in PERM:
            j = c % QC
            qc = c // QC

            @pl.when(q_x == qc)
            def _(j=j):
                if j < 2:
                    x1_in[j].wait_recv()

            @pl.when(q_z == qc)
            def _(j=j):
                if j >= 2:
                    z1_in[j].wait_recv()

            @pl.when(q_d == qc)
            def _(j=j):
                if j < 2:
                    x2_in[j].wait_recv()
                else:
                    z2_in[j - 2].wait_recv()

            val = jnp.exp(comm_ref[c].astype(f32) - m) * r
            out_store(oth_base + c * CH, val)

        for cp in ocopies:
            if cp is not None:
                cp.wait()
        srdma.wait_send()
        for j in range(QC):
            y_rdmas[j].wait_send()
            x1_out[j].wait_send()
            z1_out[j].wait_send()
        for j in range(2):
            x2_out[j].wait_send()
            z2_out[j].wait_send()

    return pl.pallas_call(
        body,
        out_shape=jax.ShapeDtypeStruct((t, 2 * v_half), jnp.bfloat16),
        in_specs=[
            pl.BlockSpec(memory_space=pltpu.VMEM),
            pl.BlockSpec(memory_space=pl.ANY),
        ],
        out_specs=pl.BlockSpec(memory_space=pl.ANY),
        scratch_shapes=[
            pltpu.VMEM((NCH, t, CH), jnp.bfloat16),
            pltpu.VMEM((NCH, t, CH), jnp.bfloat16),
            pltpu.VMEM((2, d, CH), jnp.float32),
            pltpu.VMEM((NS, t, CH), jnp.bfloat16),
            pltpu.VMEM((t, 8), jnp.float32),
            pltpu.VMEM((t, 8), jnp.float32),
            pltpu.SemaphoreType.DMA((2,)),
            pltpu.SemaphoreType.DMA((NS,)),
            pltpu.SemaphoreType.DMA((QC,)),
            pltpu.SemaphoreType.DMA((QC,)),
            pltpu.SemaphoreType.DMA((QC,)),
            pltpu.SemaphoreType.DMA((QC,)),
            pltpu.SemaphoreType.DMA((QC,)),
            pltpu.SemaphoreType.DMA((QC,)),
            pltpu.SemaphoreType.DMA((2,)),
            pltpu.SemaphoreType.DMA((2,)),
            pltpu.SemaphoreType.DMA((2,)),
            pltpu.SemaphoreType.DMA((2,)),
            pltpu.SemaphoreType.DMA,
            pltpu.SemaphoreType.DMA,
        ],
        compiler_params=pltpu.CompilerParams(
            collective_id=0,
            vmem_limit_bytes=67_000_000,
        ),
    )(x, W)


# device time: 70735 ns/iter; 1.0567x vs baseline; 1.0567x over previous
import functools

import jax
import jax.numpy as jnp
from jax import lax
from jax.experimental import pallas as pl
from jax.experimental.pallas import tpu as pltpu


def kernel(x, W):
    t, d = x.shape
    _, v_half = W.shape
    CH = 512
    NCH = v_half // CH
    QC = NCH // 4
    NS = 4
    f32 = jnp.float32
    PERM = [q * QC + j for j in range(QC) for q in range(4)]

    def body(x_ref, w_hbm, out_hbm, logits_ref, comm_ref, wbuf, ostage,
             stats_tx, stats_rx, wsems, osems, send_y, recv_y, sx1, rx1,
             sz1, rz1, sx2, rx2, sz2, rz2, stats_ssem, stats_rsem):
        my_x = lax.axis_index("x")
        my_y = lax.axis_index("y")
        my_z = lax.axis_index("z")
        ypeer = (my_x, 1 - my_y, my_z)
        xnbr = (1 - my_x, my_y, my_z)
        znbr = (my_x, my_y, 1 - my_z)
        own_base = my_y * v_half
        oth_base = (1 - my_y) * v_half
        q_mine = 2 * my_x + my_z
        q_x = 2 * (1 - my_x) + my_z
        q_z = 2 * my_x + (1 - my_z)
        q_d = 2 * (1 - my_x) + (1 - my_z)

        barrier = pltpu.get_barrier_semaphore()
        for nbr in (ypeer, xnbr, znbr):
            pl.semaphore_signal(
                barrier, inc=1, device_id=nbr,
                device_id_type=pl.DeviceIdType.MESH,
            )
        pl.semaphore_wait(barrier, 3)

        xb = x_ref[...].astype(jnp.bfloat16)

        def w_copy(i, k):
            return pltpu.make_async_copy(
                w_hbm.at[:, pl.ds(k * CH, CH)], wbuf.at[i % 2], wsems.at[i % 2]
            )

        def rdma(idx, dev, ssem, rsem):
            return pltpu.make_async_remote_copy(
                src_ref=comm_ref.at[idx],
                dst_ref=comm_ref.at[idx],
                send_sem=ssem,
                recv_sem=rsem,
                device_id=dev,
                device_id_type=pl.DeviceIdType.MESH,
            )

        y_rdmas = [
            pltpu.make_async_remote_copy(
                src_ref=logits_ref.at[q_mine * QC + j],
                dst_ref=comm_ref.at[q_mine * QC + j],
                send_sem=send_y.at[j],
                recv_sem=recv_y.at[j],
                device_id=ypeer,
                device_id_type=pl.DeviceIdType.MESH,
            )
            for j in range(QC)
        ]
        x1_out = [rdma(q_mine * QC + j, xnbr, sx1.at[j], rx1.at[j])
                  for j in range(QC)]
        z1_out = [rdma(q_mine * QC + j, znbr, sz1.at[j], rz1.at[j])
                  for j in range(QC)]
        x1_in = [rdma(q_x * QC + j, xnbr, sx1.at[j], rx1.at[j])
                 for j in range(QC)]
        z1_in = [rdma(q_z * QC + j, znbr, sz1.at[j], rz1.at[j])
                 for j in range(QC)]
        x2_out = [rdma(q_z * QC + j, xnbr, sx2.at[j], rx2.at[j])
                  for j in range(2)]
        z2_out = [rdma(q_x * QC + 2 + j, znbr, sz2.at[j], rz2.at[j])
                  for j in range(2)]
        x2_in = [rdma(q_d * QC + j, xnbr, sx2.at[j], rx2.at[j])
                 for j in range(2)]
        z2_in = [rdma(q_d * QC + 2 + j, znbr, sz2.at[j], rz2.at[j])
                 for j in range(2)]

        def relay(j):
            y_rdmas[j].wait_recv()
            x1_out[j].start()
            z1_out[j].start()

        w_copy(0, PERM[0]).start()
        m_parts = []
        for i, k in enumerate(PERM):
            if i + 1 < NCH:
                w_copy(i + 1, PERM[i + 1]).start()
            w_copy(i, k).wait()
            acc = jnp.dot(
                xb, wbuf[i % 2].astype(jnp.bfloat16),
                preferred_element_type=f32,
            )
            logits_ref[k] = acc.astype(jnp.bfloat16)
            m_parts.append(acc.max(axis=-1, keepdims=True))

            @pl.when(q_mine == k // QC)
            def _(j=k % QC):
                y_rdmas[j].start()

            if i == 8:
                relay(0)
            elif i == 12:
                relay(1)

        relay(2)
        relay(3)

        m_own = functools.reduce(jnp.maximum, m_parts)
        s_own = None
        for k in range(NCH):
            es = jnp.sum(
                jnp.exp(logits_ref[k].astype(f32) - m_own),
                axis=-1, keepdims=True,
            )
            s_own = es if s_own is None else s_own + es
        stats_tx[:, 0:1] = m_own
        stats_tx[:, 1:2] = s_own
        srdma = pltpu.make_async_remote_copy(
            src_ref=stats_tx,
            dst_ref=stats_rx,
            send_sem=stats_ssem,
            recv_sem=stats_rsem,
            device_id=ypeer,
            device_id_type=pl.DeviceIdType.MESH,
        )
        srdma.start()

        for j in range(2):
            z1_in[j].wait_recv()
            x2_out[j].start()
        for j in range(2):
            x1_in[2 + j].wait_recv()
            z2_out[j].start()

        srdma.wait_recv()
        m_oth = stats_rx[:, 0:1]
        s_oth = stats_rx[:, 1:2]
        m = jnp.maximum(m_own, m_oth)
        r = 1.0 / (s_own * jnp.exp(m_own - m) + s_oth * jnp.exp(m_oth - m))

        ocopies = [None] * NS
        ordinal = [0]

        def out_store(col, val):
            slot = ordinal[0] % NS
            if ocopies[slot] is not None:
                ocopies[slot].wait()
            ostage[slot] = val.astype(ostage.dtype)
            cp = pltpu.make_async_copy(
                ostage.at[slot], out_hbm.at[:, pl.ds(col, CH)], osems.at[slot]
            )
            cp.start()
            ocopies[slot] = cp
            ordinal[0] += 1

        for k in range(NCH):
            val = jnp.exp(logits_ref[k].astype(f32) - m) * r
            out_store(own_base + k * CH, val)
        for c in PERM:
            j = c % QC
            qc = c // QC

            @pl.when(q_x == qc)
            def _(j=j):
                if j < 2:
                    x1_in[j].wait_recv()

            @pl.when(q_z == qc)
            def _(j=j):
                if j >= 2:
                    z1_in[j].wait_recv()

            @pl.when(q_d == qc)
            def _(j=j):
                if j < 2:
                    x2_in[j].wait_recv()
                else:
                    z2_in[j - 2].wait_recv()

            val = jnp.exp(comm_ref[c].astype(f32) - m) * r
            out_store(oth_base + c * CH, val)

        for cp in ocopies:
            if cp is not None:
                cp.wait()
        srdma.wait_send()
        for j in range(QC):
            y_rdmas[j].wait_send()
            x1_out[j].wait_send()
            z1_out[j].wait_send()
        for j in range(2):
            x2_out[j].wait_send()
            z2_out[j].wait_send()

    return pl.pallas_call(
        body,
        out_shape=jax.ShapeDtypeStruct((t, 2 * v_half), jnp.bfloat16),
        in_specs=[
            pl.BlockSpec(memory_space=pltpu.VMEM),
            pl.BlockSpec(memory_space=pl.ANY),
        ],
        out_specs=pl.BlockSpec(memory_space=pl.ANY),
        scratch_shapes=[
            pltpu.VMEM((NCH, t, CH), jnp.bfloat16),
            pltpu.VMEM((NCH, t, CH), jnp.bfloat16),
            pltpu.VMEM((2, d, CH), jnp.float32),
            pltpu.VMEM((NS, t, CH), jnp.bfloat16),
            pltpu.VMEM((t, 8), jnp.float32),
            pltpu.VMEM((t, 8), jnp.float32),
            pltpu.SemaphoreType.DMA((2,)),
            pltpu.SemaphoreType.DMA((NS,)),
            pltpu.SemaphoreType.DMA((QC,)),
            pltpu.SemaphoreType.DMA((QC,)),
            pltpu.SemaphoreType.DMA((QC,)),
            pltpu.SemaphoreType.DMA((QC,)),
            pltpu.SemaphoreType.DMA((QC,)),
            pltpu.SemaphoreType.DMA((QC,)),
            pltpu.SemaphoreType.DMA((2,)),
            pltpu.SemaphoreType.DMA((2,)),
            pltpu.SemaphoreType.DMA((2,)),
            pltpu.SemaphoreType.DMA((2,)),
            pltpu.SemaphoreType.DMA,
            pltpu.SemaphoreType.DMA,
        ],
        compiler_params=pltpu.CompilerParams(
            collective_id=0,
            vmem_limit_bytes=67_000_000,
        ),
    )(x, W)


# device time: 70615 ns/iter; 1.0585x vs baseline; 1.0017x over previous
import functools

import jax
import jax.numpy as jnp
from jax import lax
from jax.experimental import pallas as pl
from jax.experimental.pallas import tpu as pltpu


def kernel(x, W):
    t, d = x.shape
    _, v_half = W.shape
    CH = 512
    NCH = v_half // CH
    QC = NCH // 4
    NS = 4
    f32 = jnp.float32
    PERM = [q * QC + j for j in range(QC) for q in range(4)]

    def body(x_ref, w_hbm, out_hbm, logits_ref, comm_ref, wbuf, ostage,
             stats_tx, stats_rx, wsems, osems, send_y, recv_y, sx1, rx1,
             sz1, rz1, sx2, rx2, sz2, rz2, stats_ssem, stats_rsem):
        my_x = lax.axis_index("x")
        my_y = lax.axis_index("y")
        my_z = lax.axis_index("z")
        ypeer = (my_x, 1 - my_y, my_z)
        xnbr = (1 - my_x, my_y, my_z)
        znbr = (my_x, my_y, 1 - my_z)
        own_base = my_y * v_half
        oth_base = (1 - my_y) * v_half
        q_mine = 2 * my_x + my_z
        q_x = 2 * (1 - my_x) + my_z
        q_z = 2 * my_x + (1 - my_z)
        q_d = 2 * (1 - my_x) + (1 - my_z)

        barrier = pltpu.get_barrier_semaphore()
        for nbr in (ypeer, xnbr, znbr):
            pl.semaphore_signal(
                barrier, inc=1, device_id=nbr,
                device_id_type=pl.DeviceIdType.MESH,
            )
        pl.semaphore_wait(barrier, 3)

        xb = x_ref[...].astype(jnp.bfloat16)

        def w_copy(i, k):
            return pltpu.make_async_copy(
                w_hbm.at[:, pl.ds(k * CH, CH)], wbuf.at[i % 2], wsems.at[i % 2]
            )

        def rdma(idx, dev, ssem, rsem):
            return pltpu.make_async_remote_copy(
                src_ref=comm_ref.at[idx],
                dst_ref=comm_ref.at[idx],
                send_sem=ssem,
                recv_sem=rsem,
                device_id=dev,
                device_id_type=pl.DeviceIdType.MESH,
            )

        y_rdmas = [
            pltpu.make_async_remote_copy(
                src_ref=logits_ref.at[q_mine * QC + j],
                dst_ref=comm_ref.at[q_mine * QC + j],
                send_sem=send_y.at[j],
                recv_sem=recv_y.at[j],
                device_id=ypeer,
                device_id_type=pl.DeviceIdType.MESH,
            )
            for j in range(QC)
        ]
        x1_out = [rdma(q_mine * QC + j, xnbr, sx1.at[j], rx1.at[j])
                  for j in range(QC)]
        z1_out = [rdma(q_mine * QC + j, znbr, sz1.at[j], rz1.at[j])
                  for j in range(QC)]
        x1_in = [rdma(q_x * QC + j, xnbr, sx1.at[j], rx1.at[j])
                 for j in range(QC)]
        z1_in = [rdma(q_z * QC + j, znbr, sz1.at[j], rz1.at[j])
                 for j in range(QC)]
        x2_out = [rdma(q_z * QC + j, xnbr, sx2.at[j], rx2.at[j])
                  for j in range(2)]
        z2_out = [rdma(q_x * QC + 2 + j, znbr, sz2.at[j], rz2.at[j])
                  for j in range(2)]
        x2_in = [rdma(q_d * QC + j, xnbr, sx2.at[j], rx2.at[j])
                 for j in range(2)]
        z2_in = [rdma(q_d * QC + 2 + j, znbr, sz2.at[j], rz2.at[j])
                 for j in range(2)]

        def relay(j):
            y_rdmas[j].wait_recv()
            x1_out[j].start()
            z1_out[j].start()

        w_copy(0, PERM[0]).start()
        m_parts = []
        for i, k in enumerate(PERM):
            if i + 1 < NCH:
                w_copy(i + 1, PERM[i + 1]).start()
            w_copy(i, k).wait()
            acc = jnp.dot(
                xb, wbuf[i % 2].astype(jnp.bfloat16),
                preferred_element_type=f32,
            )
            logits_ref[k] = acc.astype(jnp.bfloat16)
            m_parts.append(acc.max(axis=-1, keepdims=True))

            @pl.when(q_mine == k // QC)
            def _(j=k % QC):
                y_rdmas[j].start()

            if i == 8:
                relay(0)
            elif i == 12:
                relay(1)

        m_own = functools.reduce(jnp.maximum, m_parts)

        def s_half(lo, hi, s_acc):
            for k in range(lo, hi):
                es = jnp.sum(
                    jnp.exp(logits_ref[k].astype(f32) - m_own),
                    axis=-1, keepdims=True,
                )
                s_acc = es if s_acc is None else s_acc + es
            return s_acc

        relay(2)
        s_own = s_half(0, NCH // 2, None)
        relay(3)
        s_own = s_half(NCH // 2, NCH, s_own)
        stats_tx[:, 0:1] = m_own
        stats_tx[:, 1:2] = s_own
        srdma = pltpu.make_async_remote_copy(
            src_ref=stats_tx,
            dst_ref=stats_rx,
            send_sem=stats_ssem,
            recv_sem=stats_rsem,
            device_id=ypeer,
            device_id_type=pl.DeviceIdType.MESH,
        )
        srdma.start()

        for j in range(2):
            z1_in[j].wait_recv()
            x2_out[j].start()
        for j in range(2):
            x1_in[2 + j].wait_recv()
            z2_out[j].start()

        srdma.wait_recv()
        m_oth = stats_rx[:, 0:1]
        s_oth = stats_rx[:, 1:2]
        m = jnp.maximum(m_own, m_oth)
        r = 1.0 / (s_own * jnp.exp(m_own - m) + s_oth * jnp.exp(m_oth - m))

        ocopies = [None] * NS
        ordinal = [0]

        def out_store(col, val):
            slot = ordinal[0] % NS
            if ocopies[slot] is not None:
                ocopies[slot].wait()
            ostage[slot] = val.astype(ostage.dtype)
            cp = pltpu.make_async_copy(
                ostage.at[slot], out_hbm.at[:, pl.ds(col, CH)], osems.at[slot]
            )
            cp.start()
            ocopies[slot] = cp
            ordinal[0] += 1

        for k in range(NCH):
            val = jnp.exp(logits_ref[k].astype(f32) - m) * r
            out_store(own_base + k * CH, val)
        for c in PERM:
            j = c % QC
            qc = c // QC

            @pl.when(q_x == qc)
            def _(j=j):
                if j < 2:
                    x1_in[j].wait_recv()

            @pl.when(q_z == qc)
            def _(j=j):
                if j >= 2:
                    z1_in[j].wait_recv()

            @pl.when(q_d == qc)
            def _(j=j):
                if j < 2:
                    x2_in[j].wait_recv()
                else:
                    z2_in[j - 2].wait_recv()

            val = jnp.exp(comm_ref[c].astype(f32) - m) * r
            out_store(oth_base + c * CH, val)

        for cp in ocopies:
            if cp is not None:
                cp.wait()
        srdma.wait_send()
        for j in range(QC):
            y_rdmas[j].wait_send()
            x1_out[j].wait_send()
            z1_out[j].wait_send()
        for j in range(2):
            x2_out[j].wait_send()
            z2_out[j].wait_send()

    return pl.pallas_call(
        body,
        out_shape=jax.ShapeDtypeStruct((t, 2 * v_half), jnp.bfloat16),
        in_specs=[
            pl.BlockSpec(memory_space=pltpu.VMEM),
            pl.BlockSpec(memory_space=pl.ANY),
        ],
        out_specs=pl.BlockSpec(memory_space=pl.ANY),
        scratch_shapes=[
            pltpu.VMEM((NCH, t, CH), jnp.bfloat16),
            pltpu.VMEM((NCH, t, CH), jnp.bfloat16),
            pltpu.VMEM((2, d, CH), jnp.float32),
            pltpu.VMEM((NS, t, CH), jnp.bfloat16),
            pltpu.VMEM((t, 8), jnp.float32),
            pltpu.VMEM((t, 8), jnp.float32),
            pltpu.SemaphoreType.DMA((2,)),
            pltpu.SemaphoreType.DMA((NS,)),
            pltpu.SemaphoreType.DMA((QC,)),
            pltpu.SemaphoreType.DMA((QC,)),
            pltpu.SemaphoreType.DMA((QC,)),
            pltpu.SemaphoreType.DMA((QC,)),
            pltpu.SemaphoreType.DMA((QC,)),
            pltpu.SemaphoreType.DMA((QC,)),
            pltpu.SemaphoreType.DMA((2,)),
            pltpu.SemaphoreType.DMA((2,)),
            pltpu.SemaphoreType.DMA((2,)),
            pltpu.SemaphoreType.DMA((2,)),
            pltpu.SemaphoreType.DMA,
            pltpu.SemaphoreType.DMA,
        ],
        compiler_params=pltpu.CompilerParams(
            collective_id=0,
            vmem_limit_bytes=67_000_000,
        ),
    )(x, W)
